# baseline (speedup 1.0000x reference)
import functools

import jax
import jax.numpy as jnp
from jax import lax
from jax.experimental import pallas as pl
from jax.experimental.pallas import tpu as pltpu
from jax.experimental.pallas import tpu_sc as plsc

VOCAB = 1000000
D_MODEL = 64
BATCH = 4096
SEQ = 200

NC = 2
NS = 16
NW = NC * NS
N_IDX = BATCH * SEQ
CHUNK = 128
N_CHUNKS = N_IDX // CHUNK
CHUNKS_PER_W = N_CHUNKS // NW
NBUF = 4


def _embed_kernel(x_hbm, table_hbm, out_hbm, idx_v, bufs, gsems):
    wid = lax.axis_index("s") * NC + lax.axis_index("c")
    chunk0 = wid * CHUNKS_PER_W
    row0 = chunk0 * CHUNK

    pltpu.sync_copy(x_hbm.at[pl.ds(chunk0, CHUNKS_PER_W)], idx_v)

    for b in range(NBUF):
        pltpu.async_copy(table_hbm.at[idx_v.at[b]], bufs.at[b], gsems.at[b])

    def group_body(g, carry):
        for b in range(NBUF):
            j = g * NBUF + b
            pltpu.make_async_copy(
                table_hbm.at[idx_v.at[j]], bufs.at[b], gsems.at[b]
            ).wait()
            pltpu.sync_copy(bufs.at[b], out_hbm.at[pl.ds(row0 + j * CHUNK, CHUNK)])
            nxt = j + NBUF

            @pl.when(nxt < CHUNKS_PER_W)
            def _():
                pltpu.async_copy(
                    table_hbm.at[idx_v.at[nxt]], bufs.at[b], gsems.at[b]
                )

        return carry

    lax.fori_loop(0, CHUNKS_PER_W // NBUF, group_body, 0)


@jax.jit
def kernel(x, table):
    x2d = x.reshape(N_CHUNKS, CHUNK).astype(jnp.int32)
    mesh = plsc.VectorSubcoreMesh(core_axis_name="c", subcore_axis_name="s")
    run = pl.kernel(
        _embed_kernel,
        out_type=jax.ShapeDtypeStruct((N_IDX, D_MODEL), jnp.float32),
        mesh=mesh,
        scratch_types=[
            pltpu.VMEM((CHUNKS_PER_W, CHUNK), jnp.int32),
            pltpu.VMEM((NBUF, CHUNK, D_MODEL), jnp.float32),
            pltpu.SemaphoreType.DMA((NBUF,)),
        ],
        compiler_params=pltpu.CompilerParams(use_tc_tiling_on_sc=False),
    )
    out = run(x2d, table)
    return out.reshape(BATCH, SEQ, D_MODEL)
